# baseline (speedup 1.0000x reference)
import functools

import jax
import jax.numpy as jnp
from jax import lax
from jax.experimental import pallas as pl
from jax.experimental.pallas import tpu as pltpu
from jax.experimental.pallas import tpu_sc as plsc

NC, NS, L = 2, 16, 16
NW = NC * NS

EPS = 1e-7


def _rsqrt(x):
    i = lax.bitcast_convert_type(x, jnp.int32)
    i = jnp.int32(0x5F3759DF) - lax.shift_right_logical(i, 1)
    y = lax.bitcast_convert_type(i, jnp.float32)
    for _ in range(3):
        y = y * (1.5 - 0.5 * x * y * y)
    return y


def _make_kernel(B, S, V, D, P):
    assert S % NW == 0 and D % L == 0
    SPW = S // NW
    CHUNK = 32
    HB = SPW // CHUNK
    DJ = D // L

    mesh = plsc.VectorSubcoreMesh(core_axis_name="c", subcore_axis_name="s")

    @functools.partial(
        pl.kernel,
        mesh=mesh,
        compiler_params=pltpu.CompilerParams(needs_layout_passes=False),
        out_type=jax.ShapeDtypeStruct((B, S, D), jnp.float32),
        scratch_types=[
            pltpu.VMEM((B, HB, CHUNK), jnp.int32),
            pltpu.VMEM((CHUNK, D), jnp.float32),
            pltpu.VMEM((CHUNK, D), jnp.float32),
            pltpu.VMEM((D,), jnp.float32),
            pltpu.VMEM((D,), jnp.float32),
            pltpu.SemaphoreType.DMA,
        ],
    )
    def emb_kernel(ids_hbm, word_hbm, pos_hbm, gamma_hbm, beta_hbm, out_hbm,
                   idx_v, pos_v, rows_v, gam_v, bet_v, sem):
        wid = lax.axis_index("s") * NC + lax.axis_index("c")
        s0 = wid * SPW

        pltpu.sync_copy(gamma_hbm, gam_v)
        pltpu.sync_copy(beta_hbm, bet_v)
        for b in range(B):
            for h in range(HB):
                pltpu.sync_copy(ids_hbm.at[b, pl.ds(s0 + h * CHUNK, CHUNK)],
                                idx_v.at[b, h])

        inv_d = 1.0 / D

        def row_body(r, _):
            acc_s = jnp.zeros((L,), jnp.float32)
            acc_q = jnp.zeros((L,), jnp.float32)
            for j in range(DJ):
                x = rows_v[r, pl.ds(j * L, L)] + pos_v[r, pl.ds(j * L, L)]
                rows_v[r, pl.ds(j * L, L)] = x
                acc_s = acc_s + x
                acc_q = acc_q + x * x
            mean = jnp.sum(acc_s) * inv_d
            var = jnp.sum(acc_q) * inv_d - mean * mean
            rstd = _rsqrt(var + EPS)
            a = jnp.full((L,), rstd, jnp.float32)
            mb = jnp.full((L,), mean * rstd, jnp.float32)
            for j in range(DJ):
                x = rows_v[r, pl.ds(j * L, L)]
                y = (x * a - mb) * gam_v[pl.ds(j * L, L)] + bet_v[pl.ds(j * L, L)]
                rows_v[r, pl.ds(j * L, L)] = y
            return 0

        for h in range(HB):
            pltpu.sync_copy(pos_hbm.at[pl.ds(s0 + h * CHUNK, CHUNK)], pos_v)

            def batch_body(b, _):
                pltpu.async_copy(word_hbm.at[idx_v.at[b, h]], rows_v, sem).wait()
                lax.fori_loop(0, CHUNK, row_body, 0)
                pltpu.sync_copy(rows_v,
                                out_hbm.at[b, pl.ds(s0 + h * CHUNK, CHUNK)])
                return 0

            lax.fori_loop(0, B, batch_body, 0)

    return emb_kernel


def kernel(input_ids, word_emb, pos_emb, gamma, beta):
    B, S = input_ids.shape
    V, D = word_emb.shape
    P = pos_emb.shape[0]
    k = _make_kernel(B, S, V, D, P)
    return k(input_ids.astype(jnp.int32), word_emb, pos_emb, gamma, beta)
